# Initial kernel scaffold; baseline (speedup 1.0000x reference)
import jax
import jax.numpy as jnp
from jax.experimental import pallas as pl


def kernel(indices, table, W1, b1, W2, b2):
    raise NotImplementedError("write your pallas kernel here")


# speedup vs baseline: 9.0892x; 9.0892x over previous
import functools

import jax
import jax.numpy as jnp
from jax import lax
from jax.experimental import pallas as pl
from jax.experimental.pallas import tpu as pltpu
from jax.experimental.pallas import tpu_sc as plsc

_BATCH = 4096
_SEQ = 50
_EMB = 32
_TOTAL = _BATCH * _SEQ

_NC = 2
_NS = 16
_NW = _NC * _NS
_ROWS_PER_W = _TOTAL // _NW
_IDX_MINOR = 128
_GRP_PER_W = _ROWS_PER_W // _IDX_MINOR
_GRP_PER_CHUNK = 10
_N_CHUNKS = _GRP_PER_W // _GRP_PER_CHUNK
_CHUNK_ROWS = _GRP_PER_CHUNK * _IDX_MINOR


def _sc_gather(idx3d, table):
    mesh = plsc.VectorSubcoreMesh(core_axis_name="c", subcore_axis_name="s")

    @functools.partial(
        pl.kernel,
        mesh=mesh,
        out_type=jax.ShapeDtypeStruct((_TOTAL, _EMB), jnp.float32),
        scratch_types=[
            pltpu.VMEM((_GRP_PER_W, _IDX_MINOR), jnp.int32),
            pltpu.VMEM((_CHUNK_ROWS, _EMB), jnp.float32),
            pltpu.SemaphoreType.DMA,
        ],
        compiler_params=pltpu.CompilerParams(use_tc_tiling_on_sc=False),
    )
    def gather_kernel(idx_hbm, table_hbm, out_hbm, idx_v, rows_v, sem):
        wid = lax.axis_index("s") * _NC + lax.axis_index("c")
        row_base = wid * _ROWS_PER_W
        pltpu.sync_copy(idx_hbm.at[wid], idx_v)
        for c in range(_N_CHUNKS):
            copies = []
            for j in range(_GRP_PER_CHUNK):
                copies.append(pltpu.async_copy(
                    table_hbm.at[idx_v.at[c * _GRP_PER_CHUNK + j]],
                    rows_v.at[pl.ds(j * _IDX_MINOR, _IDX_MINOR)],
                    sem,
                ))
            for cp in copies:
                cp.wait()
            pltpu.sync_copy(
                rows_v,
                out_hbm.at[pl.ds(row_base + c * _CHUNK_ROWS, _CHUNK_ROWS)],
            )

    return gather_kernel(idx3d, table)


def _mlp(x, W1, b1, W2, b2):
    blk = 512

    def body(x_ref, w1_ref, b1_ref, w2_ref, b2_ref, o_ref):
        h = jnp.dot(x_ref[...], w1_ref[...], preferred_element_type=jnp.float32)
        h = jnp.maximum(h + b1_ref[...], 0.0)
        o = jnp.dot(h, w2_ref[...], preferred_element_type=jnp.float32)
        o_ref[...] = jnp.maximum(o + b2_ref[...], 0.0)

    return pl.pallas_call(
        body,
        grid=(_BATCH // blk,),
        in_specs=[
            pl.BlockSpec((blk, _SEQ * _EMB), lambda i: (i, 0)),
            pl.BlockSpec((_SEQ * _EMB, _EMB), lambda i: (0, 0)),
            pl.BlockSpec((1, _EMB), lambda i: (0, 0)),
            pl.BlockSpec((_EMB, 1), lambda i: (0, 0)),
            pl.BlockSpec((1, 1), lambda i: (0, 0)),
        ],
        out_specs=pl.BlockSpec((blk, 1), lambda i: (i, 0)),
        out_shape=jax.ShapeDtypeStruct((_BATCH, 1), jnp.float32),
    )(x, W1, b1.reshape(1, _EMB), W2, b2.reshape(1, 1))


def kernel(indices, table, W1, b1, W2, b2):
    idx3d = indices.astype(jnp.int32).reshape(_NW, _GRP_PER_W, _IDX_MINOR)
    gathered = _sc_gather(idx3d, table)
    x = gathered.reshape(_BATCH, _SEQ * _EMB)
    return _mlp(x, W1, b1, W2, b2)
